# baseline (speedup 1.0000x reference)
import functools

import jax
import jax.numpy as jnp
from jax import lax
from jax.experimental import pallas as pl
from jax.experimental.pallas import tpu as pltpu
from jax.experimental.pallas import tpu_sc as plsc

K = 16
AD = 6
CD = 16
L = 16
GC = 4
QB = 16
NQ = 8
NW = 32
BLKH = 512
HIGH = jax.lax.Precision.HIGHEST


def _lrelu(x):
    return jnp.where(x >= 0, x, 0.2 * x)


def _prep_body(aty_ref, W1_ref, b1_ref, W2_ref, b2_ref, W3_ref, b3_ref,
               at_ref):
    at = aty_ref[...]
    at = _lrelu(jnp.dot(at, W1_ref[...].T, preferred_element_type=jnp.float32) + b1_ref[...])
    at = _lrelu(jnp.dot(at, W2_ref[...].T, preferred_element_type=jnp.float32) + b2_ref[...])
    at = _lrelu(jnp.dot(at, W3_ref[...].T, preferred_element_type=jnp.float32) + b3_ref[...])
    at_ref[...] = at


def _sc_body(qx_hbm, qy_hbm, qz_hbm, ax_hbm, ay_hbm, az_hbm, at_hbm,
             watt_hbm, ss_hbm,
             axv, ayv, azv, wattv, qxv, qyv, qzv, atq, ssbuf, sem):
    wid = lax.axis_index("s") * 2 + lax.axis_index("c")
    Mp = ax_hbm.shape[0]
    per_w = qx_hbm.shape[0] // NW
    nbatches = per_w // QB
    ngroups = Mp // (GC * L)

    pltpu.sync_copy(ax_hbm, axv)
    pltpu.sync_copy(ay_hbm, ayv)
    pltpu.sync_copy(az_hbm, azv)
    pltpu.sync_copy(watt_hbm, wattv)
    watt_reg = wattv[...]
    iota = lax.iota(jnp.int32, 16)

    _gdn = lax.GatherDimensionNumbers(offset_dims=(),
                                      collapsed_slice_dims=(0,),
                                      start_index_map=(0,))

    def _lane(vec, j):
        idx = jnp.full((16, 1), j, jnp.int32)
        return lax.gather(vec, idx, _gdn, slice_sizes=(1,),
                          mode=lax.GatherScatterMode.PROMISE_IN_BOUNDS)

    def _any(m):
        cnt = plsc.all_reduce_population_count(m)
        return lax.squeeze(lax.slice(cnt, (0,), (1,)), dimensions=(0,)) > 0

    def _merge(bd, bi, dc, ic):
        rd, ri = plsc.sort_key_val(dc, ic, descending=True)
        m = bd <= rd
        nd = jnp.where(m, bd, rd)
        ni = jnp.where(m, bi, ri)
        nd, ni = plsc.sort_key_val(nd, ni)
        return nd, ni

    qbase0 = wid * per_w

    def batch_body(b, carry_b):
        qbase = qbase0 + b * QB
        pltpu.sync_copy(qx_hbm.at[pl.ds(qbase, QB)], qxv)
        pltpu.sync_copy(qy_hbm.at[pl.ds(qbase, QB)], qyv)
        pltpu.sync_copy(qz_hbm.at[pl.ds(qbase, QB)], qzv)
        qxr = qxv[...]
        qyr = qyv[...]
        qzr = qzv[...]

        def qg_body(jg, carry_q):
            j0 = jg * NQ
            qxs = [_lane(qxr, j0 + i) for i in range(NQ)]
            qys = [_lane(qyr, j0 + i) for i in range(NQ)]
            qzs = [_lane(qzr, j0 + i) for i in range(NQ)]

            def group_body(g, carry):
                bds = list(carry[0])
                bis = list(carry[1])
                base = g * (GC * L)
                for c in range(GC):
                    off = base + c * L
                    axc = axv[pl.ds(off, L)]
                    ayc = ayv[pl.ds(off, L)]
                    azc = azv[pl.ds(off, L)]
                    ic = iota + off
                    for i in range(NQ):
                        dx = axc - qxs[i]
                        dy = ayc - qys[i]
                        dz = azc - qzs[i]
                        dc = dx * dx + dy * dy + dz * dz
                        bds[i], bis[i] = _merge(bds[i], bis[i], dc, ic)
                return (tuple(bds), tuple(bis))

            init = (tuple(jnp.full((16,), 1e30, jnp.float32)
                          for _ in range(NQ)),
                    tuple(jnp.zeros((16,), jnp.int32) for _ in range(NQ)))
            bds, bis = lax.fori_loop(0, ngroups, group_body, init)

            copies = [pltpu.async_copy(at_hbm.at[bis[i]],
                                       atq.at[pl.ds(i * K, K)], sem)
                      for i in range(NQ)]
            for cp in copies:
                cp.wait()

            for i in range(NQ):
                xd = bds[i] + 1e-8
                r = 1.0 / xd
                r = r * (2.0 - xd * r)
                w = watt_reg * r
                axg = plsc.load_gather(axv, [bis[i]])
                ayg = plsc.load_gather(ayv, [bis[i]])
                azg = plsc.load_gather(azv, [bis[i]])
                u0 = w * (qxs[i] - axg)
                u1 = w * (qys[i] - ayg)
                u2 = w * (qzs[i] - azg)
                acc0 = jnp.zeros((16,), jnp.float32)
                acc1 = jnp.zeros((16,), jnp.float32)
                acc2 = jnp.zeros((16,), jnp.float32)
                for k in range(K):
                    kf = jnp.full((16,), i * K + k, jnp.int32)
                    row = plsc.load_gather(atq, [kf, iota])
                    acc0 = acc0 + _lane(u0, k) * row
                    acc1 = acc1 + _lane(u1, k) * row
                    acc2 = acc2 + _lane(u2, k) * row
                ss = acc0 * acc0 + acc1 * acc1 + acc2 * acc2
                ssbuf[pl.ds((j0 + i) * CD, CD)] = ss
            return carry_q

        lax.fori_loop(0, QB // NQ, qg_body, 0)
        pltpu.sync_copy(ssbuf, ss_hbm.at[pl.ds(qbase * CD, QB * CD)])
        return carry_b

    lax.fori_loop(0, nbatches, batch_body, 0)


def _head_body(ss_ref, W4_ref, b4_ref, W5_ref, b5_ref, W6_ref, b6_ref,
               out_ref):
    fx = jnp.sqrt(ss_ref[...])
    h = _lrelu(jnp.dot(fx, W4_ref[...].T, preferred_element_type=jnp.float32) + b4_ref[...])
    h = _lrelu(jnp.dot(h, W5_ref[...].T, preferred_element_type=jnp.float32) + b5_ref[...])
    out_ref[...] = (jnp.dot(h, W6_ref[...].T,
                            preferred_element_type=jnp.float32) + b6_ref[...])


def kernel(xyz, atom_xyz, atomtypes, batch, atom_batch,
           W1, b1, W2, b2, W3, b3, watt, W4, b4, W5, b5, W6, b6):
    N = xyz.shape[0]
    M = atom_xyz.shape[0]
    Mp = ((M + GC * L - 1) // (GC * L)) * (GC * L)
    per_w = ((N + NW * QB - 1) // (NW * QB)) * QB
    Np = NW * per_w

    Mp8 = ((Mp + 7) // 8) * 8
    aty_p = jnp.pad(atomtypes[:, :AD], ((0, Mp8 - M), (0, 0)))
    axyz_p = jnp.pad(atom_xyz, ((0, Mp - M), (0, 0)), constant_values=1e6)
    ax, ay, az = axyz_p[:, 0], axyz_p[:, 1], axyz_p[:, 2]
    x_p = jnp.pad(xyz, ((0, Np - N), (0, 0)))
    qx, qy, qz = x_p[:, 0], x_p[:, 1], x_p[:, 2]
    b1r, b2r, b3r = b1.reshape(1, CD), b2.reshape(1, CD), b3.reshape(1, CD)
    b4r, b5r, b6r = b4.reshape(1, CD), b5.reshape(1, CD), b6.reshape(1, CD)

    at = pl.pallas_call(
        _prep_body,
        out_shape=jax.ShapeDtypeStruct((Mp8, CD), jnp.float32),
    )(aty_p, W1, b1r, W2, b2r, W3, b3r)
    at = at[:Mp]

    mesh = plsc.VectorSubcoreMesh(core_axis_name="c", subcore_axis_name="s")
    sc = functools.partial(
        pl.kernel,
        out_type=jax.ShapeDtypeStruct((Np * CD,), jnp.float32),
        mesh=mesh,
        compiler_params=pltpu.CompilerParams(needs_layout_passes=False, use_tc_tiling_on_sc=False),
        scratch_types=[
            pltpu.VMEM((Mp,), jnp.float32),
            pltpu.VMEM((Mp,), jnp.float32),
            pltpu.VMEM((Mp,), jnp.float32),
            pltpu.VMEM((K,), jnp.float32),
            pltpu.VMEM((QB,), jnp.float32),
            pltpu.VMEM((QB,), jnp.float32),
            pltpu.VMEM((QB,), jnp.float32),
            pltpu.VMEM((NQ * K, CD), jnp.float32),
            pltpu.VMEM((QB * CD,), jnp.float32),
            pltpu.SemaphoreType.DMA,
        ],
    )(_sc_body)
    ss = sc(qx, qy, qz, ax, ay, az, at, watt)
    ss2 = ss.reshape(Np, CD)

    grid = (Np // BLKH,)
    full = lambda i: (0, 0)
    out = pl.pallas_call(
        _head_body,
        grid=grid,
        in_specs=[
            pl.BlockSpec((BLKH, CD), lambda i: (i, 0)),
            pl.BlockSpec((CD, CD), full),
            pl.BlockSpec((1, CD), full),
            pl.BlockSpec((CD, CD), full),
            pl.BlockSpec((1, CD), full),
            pl.BlockSpec((CD, CD), full),
            pl.BlockSpec((1, CD), full),
        ],
        out_specs=pl.BlockSpec((BLKH, CD), lambda i: (i, 0)),
        out_shape=jax.ShapeDtypeStruct((Np, CD), jnp.float32),
    )(ss2, W4, b4r, W5, b5r, W6, b6r)
    return out[:N]


# speedup vs baseline: 1.7953x; 1.7953x over previous
import functools

import jax
import jax.numpy as jnp
from jax import lax
from jax.experimental import pallas as pl
from jax.experimental.pallas import tpu as pltpu
from jax.experimental.pallas import tpu_sc as plsc

K = 16
AD = 6
CD = 16
L = 16
GC = 4
QB = 16
NQ = 4
NW = 32
BLKH = 512
HIGH = jax.lax.Precision.HIGHEST


def _lrelu(x):
    return jnp.where(x >= 0, x, 0.2 * x)


def _prep_body(aty_ref, W1_ref, b1_ref, W2_ref, b2_ref, W3_ref, b3_ref,
               at_ref):
    at = aty_ref[...]
    at = _lrelu(jnp.dot(at, W1_ref[...].T, preferred_element_type=jnp.float32) + b1_ref[...])
    at = _lrelu(jnp.dot(at, W2_ref[...].T, preferred_element_type=jnp.float32) + b2_ref[...])
    at = _lrelu(jnp.dot(at, W3_ref[...].T, preferred_element_type=jnp.float32) + b3_ref[...])
    at_ref[...] = at


def _sc_body(qx_hbm, qy_hbm, qz_hbm, ax_hbm, ay_hbm, az_hbm, at_hbm,
             watt_hbm, ss_hbm,
             axv, ayv, azv, wattv, qxv, qyv, qzv, atq, ssbuf, sem):
    wid = lax.axis_index("s") * 2 + lax.axis_index("c")
    Mp = ax_hbm.shape[0]
    per_w = qx_hbm.shape[0] // NW
    nbatches = per_w // QB
    ngroups = Mp // (GC * L)

    pltpu.sync_copy(ax_hbm, axv)
    pltpu.sync_copy(ay_hbm, ayv)
    pltpu.sync_copy(az_hbm, azv)
    pltpu.sync_copy(watt_hbm, wattv)
    watt_reg = wattv[...]
    iota = lax.iota(jnp.int32, 16)

    _gdn = lax.GatherDimensionNumbers(offset_dims=(),
                                      collapsed_slice_dims=(0,),
                                      start_index_map=(0,))

    def _lane(vec, j):
        idx = jnp.full((16, 1), j, jnp.int32)
        return lax.gather(vec, idx, _gdn, slice_sizes=(1,),
                          mode=lax.GatherScatterMode.PROMISE_IN_BOUNDS)

    def _any(m):
        cnt = plsc.all_reduce_population_count(m)
        return lax.squeeze(lax.slice(cnt, (0,), (1,)), dimensions=(0,)) > 0

    def _merge(bd, bi, dc, ic):
        rd, ri = plsc.sort_key_val(dc, ic, descending=True)
        m = bd <= rd
        nd = jnp.where(m, bd, rd)
        ni = jnp.where(m, bi, ri)
        nd, ni = plsc.sort_key_val(nd, ni)
        return nd, ni

    qbase0 = wid * per_w

    def batch_body(b, carry_b):
        qbase = qbase0 + b * QB
        pltpu.sync_copy(qx_hbm.at[pl.ds(qbase, QB)], qxv)
        pltpu.sync_copy(qy_hbm.at[pl.ds(qbase, QB)], qyv)
        pltpu.sync_copy(qz_hbm.at[pl.ds(qbase, QB)], qzv)
        qxr = qxv[...]
        qyr = qyv[...]
        qzr = qzv[...]

        def qg_body(jg, carry_q):
            j0 = jg * NQ
            qxs = [_lane(qxr, j0 + i) for i in range(NQ)]
            qys = [_lane(qyr, j0 + i) for i in range(NQ)]
            qzs = [_lane(qzr, j0 + i) for i in range(NQ)]

            def group_body(g, carry):
                bds = list(carry[0])
                bis = list(carry[1])
                base = g * (GC * L)
                for c in range(GC):
                    off = base + c * L
                    axc = axv[pl.ds(off, L)]
                    ayc = ayv[pl.ds(off, L)]
                    azc = azv[pl.ds(off, L)]
                    ic = iota + off
                    for i in range(NQ):
                        dx = axc - qxs[i]
                        dy = ayc - qys[i]
                        dz = azc - qzs[i]
                        dc = dx * dx + dy * dy + dz * dz
                        bds[i], bis[i] = _merge(bds[i], bis[i], dc, ic)
                return (tuple(bds), tuple(bis))

            init = (tuple(jnp.full((16,), 1e30, jnp.float32)
                          for _ in range(NQ)),
                    tuple(jnp.zeros((16,), jnp.int32) for _ in range(NQ)))
            bds, bis = lax.fori_loop(0, ngroups, group_body, init)

            copies = [pltpu.async_copy(at_hbm.at[bis[i]],
                                       atq.at[pl.ds(i * K, K)], sem)
                      for i in range(NQ)]
            for cp in copies:
                cp.wait()

            for i in range(NQ):
                xd = bds[i] + 1e-8
                r = 1.0 / xd
                r = r * (2.0 - xd * r)
                w = watt_reg * r
                axg = plsc.load_gather(axv, [bis[i]])
                ayg = plsc.load_gather(ayv, [bis[i]])
                azg = plsc.load_gather(azv, [bis[i]])
                u0 = w * (qxs[i] - axg)
                u1 = w * (qys[i] - ayg)
                u2 = w * (qzs[i] - azg)
                acc0 = jnp.zeros((16,), jnp.float32)
                acc1 = jnp.zeros((16,), jnp.float32)
                acc2 = jnp.zeros((16,), jnp.float32)
                for k in range(K):
                    kf = jnp.full((16,), i * K + k, jnp.int32)
                    row = plsc.load_gather(atq, [kf, iota])
                    acc0 = acc0 + _lane(u0, k) * row
                    acc1 = acc1 + _lane(u1, k) * row
                    acc2 = acc2 + _lane(u2, k) * row
                ss = acc0 * acc0 + acc1 * acc1 + acc2 * acc2
                ssbuf[pl.ds((j0 + i) * CD, CD)] = ss
            return carry_q

        lax.fori_loop(0, QB // NQ, qg_body, 0)
        pltpu.sync_copy(ssbuf, ss_hbm.at[pl.ds(qbase * CD, QB * CD)])
        return carry_b

    lax.fori_loop(0, nbatches, batch_body, 0)


def _head_body(ss_ref, W4_ref, b4_ref, W5_ref, b5_ref, W6_ref, b6_ref,
               out_ref):
    fx = jnp.sqrt(ss_ref[...])
    h = _lrelu(jnp.dot(fx, W4_ref[...].T, preferred_element_type=jnp.float32) + b4_ref[...])
    h = _lrelu(jnp.dot(h, W5_ref[...].T, preferred_element_type=jnp.float32) + b5_ref[...])
    out_ref[...] = (jnp.dot(h, W6_ref[...].T,
                            preferred_element_type=jnp.float32) + b6_ref[...])


def kernel(xyz, atom_xyz, atomtypes, batch, atom_batch,
           W1, b1, W2, b2, W3, b3, watt, W4, b4, W5, b5, W6, b6):
    N = xyz.shape[0]
    M = atom_xyz.shape[0]
    Mp = ((M + GC * L - 1) // (GC * L)) * (GC * L)
    per_w = ((N + NW * QB - 1) // (NW * QB)) * QB
    Np = NW * per_w

    Mp8 = ((Mp + 7) // 8) * 8
    aty_p = jnp.pad(atomtypes[:, :AD], ((0, Mp8 - M), (0, 0)))
    axyz_p = jnp.pad(atom_xyz, ((0, Mp - M), (0, 0)), constant_values=1e6)
    ax, ay, az = axyz_p[:, 0], axyz_p[:, 1], axyz_p[:, 2]
    x_p = jnp.pad(xyz, ((0, Np - N), (0, 0)))
    qx, qy, qz = x_p[:, 0], x_p[:, 1], x_p[:, 2]
    b1r, b2r, b3r = b1.reshape(1, CD), b2.reshape(1, CD), b3.reshape(1, CD)
    b4r, b5r, b6r = b4.reshape(1, CD), b5.reshape(1, CD), b6.reshape(1, CD)

    at = pl.pallas_call(
        _prep_body,
        out_shape=jax.ShapeDtypeStruct((Mp8, CD), jnp.float32),
    )(aty_p, W1, b1r, W2, b2r, W3, b3r)
    at = at[:Mp]

    mesh = plsc.VectorSubcoreMesh(core_axis_name="c", subcore_axis_name="s")
    sc = functools.partial(
        pl.kernel,
        out_type=jax.ShapeDtypeStruct((Np * CD,), jnp.float32),
        mesh=mesh,
        compiler_params=pltpu.CompilerParams(needs_layout_passes=False, use_tc_tiling_on_sc=False),
        scratch_types=[
            pltpu.VMEM((Mp,), jnp.float32),
            pltpu.VMEM((Mp,), jnp.float32),
            pltpu.VMEM((Mp,), jnp.float32),
            pltpu.VMEM((K,), jnp.float32),
            pltpu.VMEM((QB,), jnp.float32),
            pltpu.VMEM((QB,), jnp.float32),
            pltpu.VMEM((QB,), jnp.float32),
            pltpu.VMEM((NQ * K, CD), jnp.float32),
            pltpu.VMEM((QB * CD,), jnp.float32),
            pltpu.SemaphoreType.DMA,
        ],
    )(_sc_body)
    ss = sc(qx, qy, qz, ax, ay, az, at, watt)
    ss2 = ss.reshape(Np, CD)

    grid = (Np // BLKH,)
    full = lambda i: (0, 0)
    out = pl.pallas_call(
        _head_body,
        grid=grid,
        in_specs=[
            pl.BlockSpec((BLKH, CD), lambda i: (i, 0)),
            pl.BlockSpec((CD, CD), full),
            pl.BlockSpec((1, CD), full),
            pl.BlockSpec((CD, CD), full),
            pl.BlockSpec((1, CD), full),
            pl.BlockSpec((CD, CD), full),
            pl.BlockSpec((1, CD), full),
        ],
        out_specs=pl.BlockSpec((BLKH, CD), lambda i: (i, 0)),
        out_shape=jax.ShapeDtypeStruct((Np, CD), jnp.float32),
    )(ss2, W4, b4r, W5, b5r, W6, b6r)
    return out[:N]
